# baseline (speedup 1.0000x reference)
import functools
from math import sqrt, ceil, log

import numpy as np
import jax
import jax.numpy as jnp
from jax import lax
from jax.experimental import pallas as pl
from jax.experimental.pallas import tpu as pltpu

_FACTOR = 5
_CONSTS = {}
_U = np.uint32


def _tf2x32(k1, k2, x0, x1):
    ks0, ks1 = _U(k1), _U(k2)
    ks2 = _U(ks0 ^ ks1 ^ _U(0x1BD11BDA))
    rot0, rot1 = (13, 15, 26, 6), (17, 29, 16, 24)
    x0 = (x0.astype(_U) + ks0).astype(_U)
    x1 = (x1.astype(_U) + ks1).astype(_U)

    def rounds(a, b, rots):
        for r in rots:
            a = (a + b).astype(_U)
            b = ((b << _U(r)) | (b >> _U(32 - r))).astype(_U)
            b = (b ^ a).astype(_U)
        return a, b

    x0, x1 = rounds(x0, x1, rot0)
    x0, x1 = (x0 + ks1).astype(_U), (x1 + ks2 + _U(1)).astype(_U)
    x0, x1 = rounds(x0, x1, rot1)
    x0, x1 = (x0 + ks2).astype(_U), (x1 + ks0 + _U(2)).astype(_U)
    x0, x1 = rounds(x0, x1, rot0)
    x0, x1 = (x0 + ks0).astype(_U), (x1 + ks1 + _U(3)).astype(_U)
    x0, x1 = rounds(x0, x1, rot1)
    x0, x1 = (x0 + ks1).astype(_U), (x1 + ks2 + _U(4)).astype(_U)
    x0, x1 = rounds(x0, x1, rot0)
    x0, x1 = (x0 + ks2).astype(_U), (x1 + ks0 + _U(5)).astype(_U)
    return x0, x1


def _sample_counts(L_Q, L_K, sample_k):
    key = (L_Q, L_K, sample_k)
    if key not in _CONSTS:
        o1, o2 = _tf2x32(_U(0), _U(42), np.zeros(2, _U), np.arange(2, dtype=_U))
        n = L_Q * sample_k
        b1, b2 = _tf2x32(o1[1], o2[1], np.zeros(n, _U), np.arange(n, dtype=_U))
        idx_np = ((b1 ^ b2) % _U(L_K)).astype(np.int64).reshape(L_Q, sample_k)
        cnt = np.zeros((L_Q, L_K), np.float32)
        np.add.at(cnt, (np.arange(L_Q)[:, None], idx_np), 1.0)
        bias = np.where(cnt > 0, np.float32(0), np.float32(-np.inf))
        _CONSTS[key] = (jnp.asarray(cnt, dtype=jnp.bfloat16),
                        jnp.asarray(bias, dtype=jnp.float32))
    return _CONSTS[key]


def _body(q_ref, k_ref, v_ref, c_ref, b_ref, o_ref, ml_ref, mr_ref, gt_ref,
          *, L_Q, L_K, D, u, u_pad, qblk):
    scale = 1.0 / sqrt(D)
    n_blk = L_Q // qblk
    kmat = k_ref[0, 0]
    v = v_ref[0, 0]
    k16 = kmat.astype(jnp.bfloat16)
    v16 = v.astype(jnp.bfloat16)
    vm = jnp.sum(v, axis=0, keepdims=True) * (1.0 / L_K)

    ks = lax.dot_general(c_ref[...], k16, (((1,), (0,)), ((), ())),
                         preferred_element_type=jnp.float32)
    q16_all = q_ref[0, 0].astype(jnp.bfloat16)
    prod16 = (q16_all.astype(jnp.float32) * ks).astype(jnp.bfloat16)
    ones_d = jnp.ones((D, 1), jnp.bfloat16)
    ssum_col = lax.dot_general(prod16, ones_d, (((1,), (0,)), ((), ())),
                               preferred_element_type=jnp.float32)

    for i in range(n_blk):
        qb16 = q16_all[i * qblk:(i + 1) * qblk, :]
        s = lax.dot_general(qb16, k16, (((1,), (1,)), ((), ())),
                            preferred_element_type=jnp.float32)
        bb = b_ref[i * qblk:(i + 1) * qblk, :]
        smax = jnp.max(s + bb, axis=1)
        ssum = ssum_col[i * qblk:(i + 1) * qblk, 0]
        mv = smax - ssum * (1.0 / L_K)
        ml_ref[0:1, i * qblk:(i + 1) * qblk] = mv.reshape(1, qblk)
        mr_ref[i:i + 1, :] = mv.reshape(1, qblk)

    m_lane = ml_ref[...]
    m_t = lax.transpose(mr_ref[...], (1, 0))
    iota_lane = lax.broadcasted_iota(jnp.int32, (1, L_Q), 1)
    sub_iota = lax.broadcasted_iota(jnp.int32, (qblk, 1), 0)
    t_lane = lax.broadcasted_iota(jnp.int32, (1, u_pad), 1).astype(jnp.float32)
    for s_i in range(n_blk):
        col = m_t[:, s_i:s_i + 1]
        col_idx = sub_iota + s_i * qblk
        before = (m_lane > col) | ((m_lane == col) & (iota_lane < col_idx))
        rank = jnp.sum(jnp.where(before, 1.0, 0.0), axis=1, keepdims=True)
        onehot = jnp.where((rank == t_lane) & (rank < float(u)), 1.0, 0.0)
        gt_ref[s_i * qblk:(s_i + 1) * qblk, :] = onehot

    g16 = gt_ref[...].astype(jnp.bfloat16)
    q16 = q_ref[0, 0].astype(jnp.bfloat16)
    qr = lax.dot_general(g16, q16, (((0,), (0,)), ((), ())),
                         preferred_element_type=jnp.float32)
    s3 = lax.dot_general(qr.astype(jnp.bfloat16), k16, (((1,), (1,)), ((), ())),
                         preferred_element_type=jnp.float32) * scale
    mx = jnp.max(s3, axis=1, keepdims=True)
    e = jnp.exp(s3 - mx)
    p16 = (e / jnp.sum(e, axis=1, keepdims=True)).astype(jnp.bfloat16)
    ctx = lax.dot_general(p16, v16, (((1,), (0,)), ((), ())),
                          preferred_element_type=jnp.float32)

    upd = lax.dot_general(gt_ref[...], ctx - vm, (((1,), (0,)), ((), ())),
                          preferred_element_type=jnp.float32,
                          precision=lax.Precision.HIGHEST)
    o_ref[0, 0] = vm + upd


def kernel(queries, keys, values, attn_mask):
    B, H, L_Q, D = queries.shape
    L_K = keys.shape[2]
    Dv = values.shape[3]
    U_part = min(_FACTOR * int(ceil(log(L_K))), L_K)
    u = min(_FACTOR * int(ceil(log(L_Q))), L_Q)
    u_pad = ((u + 63) // 64) * 64
    cnt, bias = _sample_counts(L_Q, L_K, U_part)
    qblk = 128

    body = functools.partial(_body, L_Q=L_Q, L_K=L_K, D=D, u=u,
                             u_pad=u_pad, qblk=qblk)
    out = pl.pallas_call(
        body,
        grid=(B, H),
        in_specs=[
            pl.BlockSpec((1, 1, L_Q, D), lambda b, h: (b, h, 0, 0)),
            pl.BlockSpec((1, 1, L_K, D), lambda b, h: (b, h, 0, 0)),
            pl.BlockSpec((1, 1, L_K, Dv), lambda b, h: (b, h, 0, 0)),
            pl.BlockSpec((L_Q, L_K), lambda b, h: (0, 0)),
            pl.BlockSpec((L_Q, L_K), lambda b, h: (0, 0)),
        ],
        out_specs=pl.BlockSpec((1, 1, L_Q, Dv), lambda b, h: (b, h, 0, 0)),
        out_shape=jax.ShapeDtypeStruct((B, H, L_Q, Dv), jnp.float32),
        scratch_shapes=[
            pltpu.VMEM((1, L_Q), jnp.float32),
            pltpu.VMEM((L_Q // qblk, qblk), jnp.float32),
            pltpu.VMEM((L_Q, u_pad), jnp.float32),
        ],
    )(queries, keys, values, cnt, bias)
    return out


# speedup vs baseline: 1.1052x; 1.1052x over previous
import functools
from math import sqrt, ceil, log

import numpy as np
import jax
import jax.numpy as jnp
from jax import lax
from jax.experimental import pallas as pl
from jax.experimental.pallas import tpu as pltpu

_FACTOR = 5
_CONSTS = {}
_U = np.uint32


def _tf2x32(k1, k2, x0, x1):
    ks0, ks1 = _U(k1), _U(k2)
    ks2 = _U(ks0 ^ ks1 ^ _U(0x1BD11BDA))
    rot0, rot1 = (13, 15, 26, 6), (17, 29, 16, 24)
    x0 = (x0.astype(_U) + ks0).astype(_U)
    x1 = (x1.astype(_U) + ks1).astype(_U)

    def rounds(a, b, rots):
        for r in rots:
            a = (a + b).astype(_U)
            b = ((b << _U(r)) | (b >> _U(32 - r))).astype(_U)
            b = (b ^ a).astype(_U)
        return a, b

    x0, x1 = rounds(x0, x1, rot0)
    x0, x1 = (x0 + ks1).astype(_U), (x1 + ks2 + _U(1)).astype(_U)
    x0, x1 = rounds(x0, x1, rot1)
    x0, x1 = (x0 + ks2).astype(_U), (x1 + ks0 + _U(2)).astype(_U)
    x0, x1 = rounds(x0, x1, rot0)
    x0, x1 = (x0 + ks0).astype(_U), (x1 + ks1 + _U(3)).astype(_U)
    x0, x1 = rounds(x0, x1, rot1)
    x0, x1 = (x0 + ks1).astype(_U), (x1 + ks2 + _U(4)).astype(_U)
    x0, x1 = rounds(x0, x1, rot0)
    x0, x1 = (x0 + ks2).astype(_U), (x1 + ks0 + _U(5)).astype(_U)
    return x0, x1


def _sample_counts(L_Q, L_K, sample_k):
    key = (L_Q, L_K, sample_k)
    if key not in _CONSTS:
        o1, o2 = _tf2x32(_U(0), _U(42), np.zeros(2, _U), np.arange(2, dtype=_U))
        n = L_Q * sample_k
        b1, b2 = _tf2x32(o1[1], o2[1], np.zeros(n, _U), np.arange(n, dtype=_U))
        idx_np = ((b1 ^ b2) % _U(L_K)).astype(np.int64).reshape(L_Q, sample_k)
        cnt = np.zeros((L_Q, L_K), np.float32)
        np.add.at(cnt, (np.arange(L_Q)[:, None], idx_np), 1.0)
        bias = np.where(cnt > 0, np.float32(0), np.float32(-np.inf))
        _CONSTS[key] = (jnp.asarray(cnt, dtype=jnp.bfloat16),
                        jnp.asarray(bias, dtype=jnp.float32))
    return _CONSTS[key]


def _body(q_ref, k_ref, v_ref, c_ref, o_ref, ml_ref, mr_ref, gt_ref,
          *, L_Q, L_K, D, u, u_pad, qblk):
    scale = 1.0 / sqrt(D)
    n_blk = L_Q // qblk
    kmat = k_ref[0, 0]
    v = v_ref[0, 0]
    k16 = kmat.astype(jnp.bfloat16)
    v16 = v.astype(jnp.bfloat16)
    vm = jnp.sum(v, axis=0, keepdims=True) * (1.0 / L_K)

    ks = lax.dot_general(c_ref[...], k16, (((1,), (0,)), ((), ())),
                         preferred_element_type=jnp.float32)

    for i in range(n_blk):
        qb = q_ref[0, 0, i * qblk:(i + 1) * qblk, :]
        qb16 = qb.astype(jnp.bfloat16)
        s = lax.dot_general(qb16, k16, (((1,), (1,)), ((), ())),
                            preferred_element_type=jnp.float32)
        cb = c_ref[i * qblk:(i + 1) * qblk, :]
        smax = jnp.max(jnp.where(cb > 0, s, -jnp.inf), axis=1)
        qf = qb16.astype(jnp.float32)
        ssum = jnp.sum(qf * ks[i * qblk:(i + 1) * qblk, :], axis=1)
        mv = smax - ssum * (1.0 / L_K)
        ml_ref[0:1, i * qblk:(i + 1) * qblk] = mv.reshape(1, qblk)
        mr_ref[i:i + 1, :] = mv.reshape(1, qblk)

    m_lane = ml_ref[...]
    m_t = lax.transpose(mr_ref[...], (1, 0))
    iota_lane = lax.broadcasted_iota(jnp.int32, (1, L_Q), 1)
    sub_iota = lax.broadcasted_iota(jnp.int32, (qblk, 1), 0)
    t_lane = lax.broadcasted_iota(jnp.int32, (1, u_pad), 1).astype(jnp.float32)
    for s_i in range(n_blk):
        col = m_t[:, s_i:s_i + 1]
        col_idx = sub_iota + s_i * qblk
        before = (m_lane > col) | ((m_lane == col) & (iota_lane < col_idx))
        rank = jnp.sum(jnp.where(before, 1.0, 0.0), axis=1, keepdims=True)
        onehot = jnp.where((rank == t_lane) & (rank < float(u)), 1.0, 0.0)
        gt_ref[s_i * qblk:(s_i + 1) * qblk, :] = onehot

    g16 = gt_ref[...].astype(jnp.bfloat16)
    q16 = q_ref[0, 0].astype(jnp.bfloat16)
    qr = lax.dot_general(g16, q16, (((0,), (0,)), ((), ())),
                         preferred_element_type=jnp.float32)
    s3 = lax.dot_general(qr.astype(jnp.bfloat16), k16, (((1,), (1,)), ((), ())),
                         preferred_element_type=jnp.float32) * scale
    mx = jnp.max(s3, axis=1, keepdims=True)
    e = jnp.exp(s3 - mx)
    p16 = (e / jnp.sum(e, axis=1, keepdims=True)).astype(jnp.bfloat16)
    ctx = lax.dot_general(p16, v16, (((1,), (0,)), ((), ())),
                          preferred_element_type=jnp.float32)

    upd = lax.dot_general(gt_ref[...], ctx - vm, (((1,), (0,)), ((), ())),
                          preferred_element_type=jnp.float32,
                          precision=lax.Precision.HIGHEST)
    o_ref[0, 0] = vm + upd


def kernel(queries, keys, values, attn_mask):
    B, H, L_Q, D = queries.shape
    L_K = keys.shape[2]
    Dv = values.shape[3]
    U_part = min(_FACTOR * int(ceil(log(L_K))), L_K)
    u = min(_FACTOR * int(ceil(log(L_Q))), L_Q)
    u_pad = ((u + 63) // 64) * 64
    cnt, bias = _sample_counts(L_Q, L_K, U_part)
    qblk = 128

    body = functools.partial(_body, L_Q=L_Q, L_K=L_K, D=D, u=u,
                             u_pad=u_pad, qblk=qblk)
    out = pl.pallas_call(
        body,
        grid=(B, H),
        in_specs=[
            pl.BlockSpec((1, 1, L_Q, D), lambda b, h: (b, h, 0, 0)),
            pl.BlockSpec((1, 1, L_K, D), lambda b, h: (b, h, 0, 0)),
            pl.BlockSpec((1, 1, L_K, Dv), lambda b, h: (b, h, 0, 0)),
            pl.BlockSpec((L_Q, L_K), lambda b, h: (0, 0)),
        ],
        out_specs=pl.BlockSpec((1, 1, L_Q, Dv), lambda b, h: (b, h, 0, 0)),
        out_shape=jax.ShapeDtypeStruct((B, H, L_Q, Dv), jnp.float32),
        scratch_shapes=[
            pltpu.VMEM((1, L_Q), jnp.float32),
            pltpu.VMEM((L_Q // qblk, qblk), jnp.float32),
            pltpu.VMEM((L_Q, u_pad), jnp.float32),
        ],
    )(queries, keys, values, cnt)
    return out


# speedup vs baseline: 1.2187x; 1.1027x over previous
import functools
from math import sqrt, ceil, log

import numpy as np
import jax
import jax.numpy as jnp
from jax import lax
from jax.experimental import pallas as pl
from jax.experimental.pallas import tpu as pltpu

_FACTOR = 5
_CONSTS = {}
_U = np.uint32


def _tf2x32(k1, k2, x0, x1):
    ks0, ks1 = _U(k1), _U(k2)
    ks2 = _U(ks0 ^ ks1 ^ _U(0x1BD11BDA))
    rot0, rot1 = (13, 15, 26, 6), (17, 29, 16, 24)
    x0 = (x0.astype(_U) + ks0).astype(_U)
    x1 = (x1.astype(_U) + ks1).astype(_U)

    def rounds(a, b, rots):
        for r in rots:
            a = (a + b).astype(_U)
            b = ((b << _U(r)) | (b >> _U(32 - r))).astype(_U)
            b = (b ^ a).astype(_U)
        return a, b

    x0, x1 = rounds(x0, x1, rot0)
    x0, x1 = (x0 + ks1).astype(_U), (x1 + ks2 + _U(1)).astype(_U)
    x0, x1 = rounds(x0, x1, rot1)
    x0, x1 = (x0 + ks2).astype(_U), (x1 + ks0 + _U(2)).astype(_U)
    x0, x1 = rounds(x0, x1, rot0)
    x0, x1 = (x0 + ks0).astype(_U), (x1 + ks1 + _U(3)).astype(_U)
    x0, x1 = rounds(x0, x1, rot1)
    x0, x1 = (x0 + ks1).astype(_U), (x1 + ks2 + _U(4)).astype(_U)
    x0, x1 = rounds(x0, x1, rot0)
    x0, x1 = (x0 + ks2).astype(_U), (x1 + ks0 + _U(5)).astype(_U)
    return x0, x1


def _sample_counts(L_Q, L_K, sample_k):
    key = (L_Q, L_K, sample_k)
    if key not in _CONSTS:
        o1, o2 = _tf2x32(_U(0), _U(42), np.zeros(2, _U), np.arange(2, dtype=_U))
        n = L_Q * sample_k
        b1, b2 = _tf2x32(o1[1], o2[1], np.zeros(n, _U), np.arange(n, dtype=_U))
        idx_np = ((b1 ^ b2) % _U(L_K)).astype(np.int64).reshape(L_Q, sample_k)
        cnt = np.zeros((L_Q, L_K), np.float32)
        np.add.at(cnt, (np.arange(L_Q)[:, None], idx_np), 1.0)
        bias = np.where(cnt > 0, np.float32(0), np.float32(-np.inf))
        _CONSTS[key] = (jnp.asarray(cnt, dtype=jnp.bfloat16),
                        jnp.asarray(bias, dtype=jnp.float32))
    return _CONSTS[key]


def _body(q_ref, k_ref, v_ref, c_ref, o_ref, ml_ref, mr_ref, gt_ref,
          *, L_Q, L_K, D, u, u_pad, qblk):
    scale = 1.0 / sqrt(D)
    n_blk = L_Q // qblk
    kmat = k_ref[0, 0]
    v = v_ref[0, 0]
    k16 = kmat.astype(jnp.bfloat16)
    v16 = v.astype(jnp.bfloat16)
    vm = jnp.sum(v, axis=0, keepdims=True) * (1.0 / L_K)

    ks = lax.dot_general(c_ref[...], k16, (((1,), (0,)), ((), ())),
                         preferred_element_type=jnp.float32)

    p1blk = 2 * qblk
    for i in range(L_Q // p1blk):
        qb = q_ref[0, 0, i * p1blk:(i + 1) * p1blk, :]
        qb16 = qb.astype(jnp.bfloat16)
        s = lax.dot_general(qb16, k16, (((1,), (1,)), ((), ())),
                            preferred_element_type=jnp.float32)
        cb = c_ref[i * p1blk:(i + 1) * p1blk, :]
        smax = jnp.max(jnp.where(cb > 0, s, -jnp.inf), axis=1)
        qf = qb16.astype(jnp.float32)
        ssum = jnp.sum(qf * ks[i * p1blk:(i + 1) * p1blk, :], axis=1)
        mv = smax - ssum * (1.0 / L_K)
        ml_ref[0:1, i * p1blk:(i + 1) * p1blk] = mv.reshape(1, p1blk)
        mr_ref[2 * i:2 * i + 2, :] = mv.reshape(2, qblk)

    m_lane = ml_ref[...]
    m_t = lax.transpose(mr_ref[...], (1, 0))
    sub_iota = lax.broadcasted_iota(jnp.int32, (qblk, 1), 0)
    diag_iota = lax.broadcasted_iota(jnp.int32, (1, qblk), 1)
    t_lane = lax.broadcasted_iota(jnp.int32, (1, u_pad), 1).astype(jnp.float32)
    for s_i in range(n_blk):
        col = m_t[:, s_i:s_i + 1]
        lo, hi = s_i * qblk, (s_i + 1) * qblk
        rank = jnp.zeros((qblk, 1), jnp.float32)
        if s_i > 0:
            left = jnp.where(m_lane[:, :lo] >= col, 1.0, 0.0)
            rank = rank + jnp.sum(left, axis=1, keepdims=True)
        if s_i < n_blk - 1:
            right = jnp.where(m_lane[:, hi:] > col, 1.0, 0.0)
            rank = rank + jnp.sum(right, axis=1, keepdims=True)
        md = m_lane[:, lo:hi]
        diag = (md > col) | ((md == col) & (diag_iota < sub_iota))
        rank = rank + jnp.sum(jnp.where(diag, 1.0, 0.0), axis=1, keepdims=True)
        onehot = jnp.where((rank == t_lane) & (rank < float(u)), 1.0, 0.0)
        gt_ref[lo:hi, :] = onehot

    g16 = gt_ref[...].astype(jnp.bfloat16)
    q16 = q_ref[0, 0].astype(jnp.bfloat16)
    qr = lax.dot_general(g16, q16, (((0,), (0,)), ((), ())),
                         preferred_element_type=jnp.float32)
    s3 = lax.dot_general(qr.astype(jnp.bfloat16), k16, (((1,), (1,)), ((), ())),
                         preferred_element_type=jnp.float32) * scale
    mx = jnp.max(s3, axis=1, keepdims=True)
    e = jnp.exp(s3 - mx)
    p16 = (e / jnp.sum(e, axis=1, keepdims=True)).astype(jnp.bfloat16)
    ctx = lax.dot_general(p16, v16, (((1,), (0,)), ((), ())),
                          preferred_element_type=jnp.float32)

    upd = lax.dot_general(gt_ref[...], ctx - vm, (((1,), (0,)), ((), ())),
                          preferred_element_type=jnp.float32,
                          precision=lax.Precision.HIGHEST)
    o_ref[0, 0] = vm + upd


def kernel(queries, keys, values, attn_mask):
    B, H, L_Q, D = queries.shape
    L_K = keys.shape[2]
    Dv = values.shape[3]
    U_part = min(_FACTOR * int(ceil(log(L_K))), L_K)
    u = min(_FACTOR * int(ceil(log(L_Q))), L_Q)
    u_pad = ((u + 63) // 64) * 64
    cnt, bias = _sample_counts(L_Q, L_K, U_part)
    qblk = 128

    body = functools.partial(_body, L_Q=L_Q, L_K=L_K, D=D, u=u,
                             u_pad=u_pad, qblk=qblk)
    out = pl.pallas_call(
        body,
        grid=(B, H),
        in_specs=[
            pl.BlockSpec((1, 1, L_Q, D), lambda b, h: (b, h, 0, 0)),
            pl.BlockSpec((1, 1, L_K, D), lambda b, h: (b, h, 0, 0)),
            pl.BlockSpec((1, 1, L_K, Dv), lambda b, h: (b, h, 0, 0)),
            pl.BlockSpec((L_Q, L_K), lambda b, h: (0, 0)),
        ],
        out_specs=pl.BlockSpec((1, 1, L_Q, Dv), lambda b, h: (b, h, 0, 0)),
        out_shape=jax.ShapeDtypeStruct((B, H, L_Q, Dv), jnp.float32),
        scratch_shapes=[
            pltpu.VMEM((1, L_Q), jnp.float32),
            pltpu.VMEM((L_Q // qblk, qblk), jnp.float32),
            pltpu.VMEM((L_Q, u_pad), jnp.float32),
        ],
    )(queries, keys, values, cnt)
    return out
